# baseline (speedup 1.0000x reference)
import functools

import jax
import jax.numpy as jnp
from jax import lax
from jax.experimental import pallas as pl
from jax.experimental.pallas import tpu as pltpu
from jax.experimental.pallas import tpu_sc as plsc

N = 50000
E = 800000
G = 64
D1 = 32

NC, NS, L = 2, 16, 16
NW = NC * NS

CHUNK = 128
NCW = 400
KC = 20
TOT_CH = NS * NCW
E_PAD = TOT_CH * CHUNK
FAST_CID = 0
ROWS_PER_TILE = 3200
N_ACC = NS * ROWS_PER_TILE

PR = 1568
N_POOL = NW * PR
GP = G + 1

BLK = 2000
NBLK = N // BLK

_MESH = plsc.VectorSubcoreMesh(
    core_axis_name="c", subcore_axis_name="s", num_cores=NC, num_subcores=NS
)
_SC_PARAMS = pltpu.CompilerParams(use_tc_tiling_on_sc=False)


def _make_sc_agg_pair(d, ncw_f, ncw_s):
    ng_f, ng_s = ncw_f // KC, ncw_s // KC

    @functools.partial(
        pl.kernel,
        out_type=jax.ShapeDtypeStruct((2, NC, N_ACC, d), jnp.float32),
        mesh=_MESH,
        compiler_params=_SC_PARAMS,
        scratch_types=[
            pltpu.VMEM((KC, CHUNK), jnp.int32),
            pltpu.VMEM((KC, CHUNK), jnp.int32),
            pltpu.VMEM((KC, CHUNK), jnp.int32),
            pltpu.VMEM((KC, CHUNK), jnp.int32),
            pltpu.VMEM((CHUNK, d), jnp.float32),
            pltpu.VMEM((CHUNK, d), jnp.float32),
            pltpu.VMEM((CHUNK, d), jnp.float32),
            pltpu.VMEM((CHUNK, d), jnp.float32),
            pltpu.VMEM_SHARED((N_ACC, d), jnp.float32),
            pltpu.SemaphoreType.DMA,
            pltpu.SemaphoreType.DMA,
            pltpu.SemaphoreType.DMA,
            pltpu.SemaphoreType.DMA,
            pltpu.SemaphoreType.DMA,
            pltpu.SemaphoreType.DMA,
            pltpu.SemaphoreType.DMA,
            pltpu.SemaphoreType.DMA,
            pltpu.SemaphoreType.DMA,
        ],
    )
    def agg(h_hbm, srcs_hbm, dsts_hbm, zer_hbm, out_hbm,
            src_vA, dst_vA, src_vB, dst_vB, b0, b1, b2, b3, acc,
            g0s, g1s, g2s, g3s, s0s, s1s, s2s, s3s, isem):
        cid = lax.axis_index("c")
        sid = lax.axis_index("s")
        is_fast = cid == FAST_CID
        cbase = jnp.where(is_fast, sid * ncw_f, NS * ncw_f + sid * ncw_s)
        npair = jnp.where(is_fast, ng_f // 2, ng_s // 2)
        ng = 2 * npair
        bufs = (b0, b1, b2, b3)
        gsems = (g0s, g1s, g2s, g3s)
        ssems = (s0s, s1s, s2s, s3s)
        myrows = pl.ds(sid * ROWS_PER_TILE, ROWS_PER_TILE)

        def phase(br):
            src_hbm = srcs_hbm.at[br]
            dst_hbm = dsts_hbm.at[br]

            pltpu.sync_copy(zer_hbm, acc.at[myrows, :])
            plsc.subcore_barrier()

            def process_chunks(src_v, dst_v):
                for b in range(4):
                    pltpu.async_copy(h_hbm.at[src_v.at[b]], bufs[b], gsems[b])

                def qbody(t, _):
                    jb = 4 * t
                    for b in range(4):
                        j = jb + b
                        pltpu.make_async_copy(
                            h_hbm.at[src_v.at[j]], bufs[b], gsems[b]
                        ).wait()
                        pltpu.async_copy(
                            bufs[b], acc.at[dst_v.at[j]], ssems[b], add=True
                        )
                    for b in range(4):
                        nj = jb + b + 4

                        @pl.when(nj < KC)
                        def _(b=b, nj=nj, jb=jb):
                            pltpu.make_async_copy(
                                bufs[b], acc.at[dst_v.at[jb + b]], ssems[b]
                            ).wait()
                            pltpu.async_copy(
                                h_hbm.at[src_v.at[nj]], bufs[b], gsems[b]
                            )

                    return 0

                lax.fori_loop(0, KC // 4, qbody, 0)
                for b in range(4):
                    pltpu.make_async_copy(
                        bufs[b], acc.at[dst_v.at[KC - 4 + b]], ssems[b]
                    ).wait()

            def stage(g, sv, dv, sem):
                pltpu.async_copy(src_hbm.at[pl.ds(cbase + g * KC, KC)], sv, sem)
                pltpu.async_copy(dst_hbm.at[pl.ds(cbase + g * KC, KC)], dv, sem)

            def stage_wait(g, sv, dv, sem):
                pltpu.make_async_copy(
                    src_hbm.at[pl.ds(cbase + g * KC, KC)], sv, sem
                ).wait()
                pltpu.make_async_copy(
                    dst_hbm.at[pl.ds(cbase + g * KC, KC)], dv, sem
                ).wait()

            pltpu.sync_copy(src_hbm.at[pl.ds(cbase, KC)], src_vA)
            pltpu.sync_copy(dst_hbm.at[pl.ds(cbase, KC)], dst_vA)

            def gpair(q, _):
                ga = 2 * q
                stage(ga + 1, src_vB, dst_vB, isem)
                process_chunks(src_vA, dst_vA)
                stage_wait(ga + 1, src_vB, dst_vB, isem)

                @pl.when(ga + 2 < ng)
                def _():
                    stage(ga + 2, src_vA, dst_vA, isem)

                process_chunks(src_vB, dst_vB)

                @pl.when(ga + 2 < ng)
                def _():
                    stage_wait(ga + 2, src_vA, dst_vA, isem)

                return 0

            lax.fori_loop(0, npair, gpair, 0)
            plsc.subcore_barrier()

            pltpu.sync_copy(acc.at[myrows, :], out_hbm.at[br, cid, myrows, :])

        phase(0)
        phase(1)

    return agg


_sc_agg8 = _make_sc_agg_pair(8, 240, 160)
_sc_agg32 = _make_sc_agg_pair(32, 280, 120)


@functools.partial(
    pl.kernel,
    out_type=jax.ShapeDtypeStruct((2, NW, GP, D1), jnp.float32),
    mesh=_MESH,
    compiler_params=_SC_PARAMS,
    scratch_types=[
        pltpu.VMEM((PR,), jnp.int32),
        pltpu.VMEM((PR, D1), jnp.float32),
        pltpu.VMEM((GP, D1), jnp.float32),
    ],
)
def _sc_pool_pair(hb_hbm, hr_hbm, bb_hbm, br_hbm, out_hbm, batch_v, h_v, acc):
    cid = lax.axis_index("c")
    sid = lax.axis_index("s")
    wid = cid * NS + sid
    base = wid * PR
    neg = jnp.full((L,), -3.4e38, jnp.float32)

    def phase(br, h_hbm, b_hbm):
        pltpu.sync_copy(b_hbm.at[pl.ds(base, PR)], batch_v)
        pltpu.sync_copy(h_hbm.at[pl.ds(base, PR), :], h_v)

        def ini(g, _):
            acc[g, pl.ds(0, L)] = neg
            acc[g, pl.ds(L, L)] = neg
            return 0

        lax.fori_loop(0, GP, ini, 0)

        def body(q, _):
            gvec = batch_v[pl.ds(q * L, L)]
            for lane in range(L):
                r = q * L + lane
                g = gvec[lane]
                acc[g, pl.ds(0, L)] = jnp.maximum(
                    acc[g, pl.ds(0, L)], h_v[r, pl.ds(0, L)]
                )
                acc[g, pl.ds(L, L)] = jnp.maximum(
                    acc[g, pl.ds(L, L)], h_v[r, pl.ds(L, L)]
                )
            return 0

        lax.fori_loop(0, PR // L, body, 0)
        pltpu.sync_copy(acc, out_hbm.at[br, wid])

    phase(0, hb_hbm, bb_hbm)
    phase(1, hr_hbm, br_hbm)


def _make_layer(d):
    def body(h_ref, agg_ref, w1, b1, w2, b2, t_ref, st_ref, accs):
        i = pl.program_id(1)
        u = h_ref[...] + agg_ref[0, 0] + agg_ref[0, 1]
        t = jnp.dot(u, w1[0], preferred_element_type=jnp.float32) + b1[0]
        t = jnp.maximum(t, 0.0)
        t = jnp.dot(t, w2[0], preferred_element_type=jnp.float32) + b2[0]
        t = jnp.maximum(t, 0.0)
        t_ref[...] = t

        @pl.when(i == 0)
        def _():
            accs[...] = jnp.zeros_like(accs)

        s = jnp.sum(t, axis=0, keepdims=True)
        m = s * (1.0 / BLK)
        d2 = t - m
        accs[0:1, :] += s
        accs[1:2, :] += jnp.sum(d2 * d2, axis=0, keepdims=True)
        accs[2:3, :] += m * m

        @pl.when(i == NBLK - 1)
        def _():
            st_ref[...] = accs[...].reshape(1, 8, D1)

    return pl.pallas_call(
        body,
        grid=(2, NBLK),
        in_specs=[
            pl.BlockSpec((BLK, d), lambda b, i: (b * NBLK + i, 0)),
            pl.BlockSpec((1, NC, BLK, d), lambda b, i: (b, 0, i, 0)),
            pl.BlockSpec((1, d, D1), lambda b, i: (b, 0, 0)),
            pl.BlockSpec((1, 1, D1), lambda b, i: (b, 0, 0)),
            pl.BlockSpec((1, D1, D1), lambda b, i: (b, 0, 0)),
            pl.BlockSpec((1, 1, D1), lambda b, i: (b, 0, 0)),
        ],
        out_specs=[
            pl.BlockSpec((BLK, D1), lambda b, i: (b * NBLK + i, 0)),
            pl.BlockSpec((1, 8, D1), lambda b, i: (b, 0, 0)),
        ],
        out_shape=[
            jax.ShapeDtypeStruct((2 * N, D1), jnp.float32),
            jax.ShapeDtypeStruct((2, 8, D1), jnp.float32),
        ],
        scratch_shapes=[pltpu.VMEM((8, D1), jnp.float32)],
    )


_layer8 = _make_layer(8)
_layer32 = _make_layer(32)


def _norm_pair():
    def body(t_ref, s_ref, c_ref, o_ref):
        o_ref[...] = t_ref[...] * s_ref[0] + c_ref[0]

    return pl.pallas_call(
        body,
        grid=(2, NBLK),
        in_specs=[
            pl.BlockSpec((BLK, D1), lambda b, i: (b * NBLK + i, 0)),
            pl.BlockSpec((1, 1, D1), lambda b, i: (b, 0, 0)),
            pl.BlockSpec((1, 1, D1), lambda b, i: (b, 0, 0)),
        ],
        out_specs=pl.BlockSpec((BLK, D1), lambda b, i: (b * NBLK + i, 0)),
        out_shape=jax.ShapeDtypeStruct((2 * N, D1), jnp.float32),
    )


_norm2 = _norm_pair()


def _make_norm3(boff):
    def body(t_ref, s_ref, c_ref, o_ref):
        o_ref[...] = t_ref[...] * s_ref[0] + c_ref[0]

    return pl.pallas_call(
        body,
        grid=(NBLK,),
        in_specs=[
            pl.BlockSpec((BLK, D1), lambda i: (boff * NBLK + i, 0)),
            pl.BlockSpec((1, 1, D1), lambda i: (boff, 0, 0)),
            pl.BlockSpec((1, 1, D1), lambda i: (boff, 0, 0)),
        ],
        out_specs=pl.BlockSpec((BLK, D1), lambda i: (i, 0)),
        out_shape=jax.ShapeDtypeStruct((N_POOL, D1), jnp.float32),
    )


_norm3_b = _make_norm3(0)
_norm3_r = _make_norm3(1)


def _head(pm, wb, bb, wr, br2, wbe, bbe, wm, bm):
    def body(pm_ref, wb_r, bb_r, wr_r, br_r, wbe_r, bbe_r, wm_r, bm_r, o_ref):
        pb = jnp.max(pm_ref[0], axis=0)[:G, :]
        eb = jnp.maximum(
            jnp.dot(pb, wb_r[...], preferred_element_type=jnp.float32) + bb_r[...], 0.0
        )
        pr = jnp.max(pm_ref[1], axis=0)[:G, :]
        er = jnp.maximum(
            jnp.dot(pr, wr_r[...], preferred_element_type=jnp.float32) + br_r[...], 0.0
        )
        cat = jnp.concatenate([eb, er], axis=-1)
        h = jnp.maximum(
            jnp.dot(cat, wbe_r[...], preferred_element_type=jnp.float32) + bbe_r[...],
            0.0,
        )
        z = jnp.dot(h, wm_r[...], preferred_element_type=jnp.float32) + bm_r[...]
        o_ref[...] = 1.0 / (1.0 + jnp.exp(-z))

    return pl.pallas_call(
        body,
        out_shape=jax.ShapeDtypeStruct((G, 1), jnp.float32),
    )(pm, wb, bb, wr, br2, wbe, bbe, wm, bm)


def _prep_edges(ei, src_off):
    src = jnp.concatenate([ei[0] + src_off, jnp.full((E_PAD - E,), src_off, jnp.int32)])
    pad_dst = N + (jnp.arange(E_PAD - E, dtype=jnp.int32) % (N_ACC - N))
    dst = jnp.concatenate([ei[1], pad_dst])
    return src.reshape(TOT_CH, CHUNK), dst.reshape(TOT_CH, CHUNK)


def _stack2(pb, pr):
    return jnp.stack([pb, pr])


def kernel(data_base, edge_index_base, batch_base,
           data_residual, edge_index_residual, batch_residual, params):
    p = params
    src_b, dst_b = _prep_edges(edge_index_base, 0)
    src_r, dst_r = _prep_edges(edge_index_residual, N)
    srcs = _stack2(src_b, src_r)
    dsts = _stack2(dst_b, dst_r)
    batch_b = jnp.pad(batch_base, (0, N_POOL - N), constant_values=G).astype(jnp.int32)
    batch_r = jnp.pad(batch_residual, (0, N_POOL - N), constant_values=G).astype(
        jnp.int32
    )
    zer8 = jnp.zeros((ROWS_PER_TILE, 8), jnp.float32)
    zer32 = jnp.zeros((ROWS_PER_TILE, D1), jnp.float32)

    x2 = jnp.stack([data_base, data_residual])
    h = jnp.pad(x2, ((0, 0), (0, 0), (0, 2))).reshape(2 * N, 8)

    for i in range(1, 4):
        d = 8 if i == 1 else D1
        w1 = _stack2(p["base_c%d_W1" % i], p["res_c%d_W1" % i])
        if i == 1:
            w1 = jnp.pad(w1, ((0, 0), (0, 8 - w1.shape[1]), (0, 0)))
        b1 = _stack2(p["base_c%d_b1" % i], p["res_c%d_b1" % i]).reshape(2, 1, D1)
        w2 = _stack2(p["base_c%d_W2" % i], p["res_c%d_W2" % i])
        b2 = _stack2(p["base_c%d_b2" % i], p["res_c%d_b2" % i]).reshape(2, 1, D1)
        g2 = _stack2(p["base_bn%d_g" % i], p["res_bn%d_g" % i])
        bb2 = _stack2(p["base_bn%d_b" % i], p["res_bn%d_b" % i])

        agg = (_sc_agg8 if d == 8 else _sc_agg32)(
            h, srcs, dsts, zer8 if d == 8 else zer32
        )
        t, st = (_layer8 if d == 8 else _layer32)(h, agg, w1, b1, w2, b2)
        mean = st[:, 0] / N
        var = st[:, 1] / N + (st[:, 2] / NBLK - mean * mean)
        s = g2 * lax.rsqrt(var + 1e-5)
        c = bb2 - mean * s
        s = s.reshape(2, 1, D1)
        c = c.reshape(2, 1, D1)
        if i < 3:
            h = _norm2(t, s, c)
        else:
            h3_b = _norm3_b(t, s, c)
            h3_r = _norm3_r(t, s, c)

    pm = _sc_pool_pair(h3_b, h3_r, batch_b, batch_r)
    return _head(
        pm,
        p["base_Wbr"], p["base_bbr"].reshape(1, D1),
        p["res_Wbr"], p["res_bbr"].reshape(1, D1),
        p["W_before"], p["b_before"].reshape(1, 16),
        p["W_mean"], p["b_mean"].reshape(1, 1),
    )


# speedup vs baseline: 1.9635x; 1.9635x over previous
import functools

import jax
import jax.numpy as jnp
from jax import lax
from jax.experimental import pallas as pl
from jax.experimental.pallas import tpu as pltpu
from jax.experimental.pallas import tpu_sc as plsc

N = 50000
E = 800000
G = 64
D1 = 32
DH = 16

NC, NS, L = 2, 16, 16
NW = NC * NS

CHUNK = 128
NCW = 400
KC = 20
TOT_CH = NS * NCW
E_PAD = TOT_CH * CHUNK
ROWS_PER_TILE = 3200
N_ACC = NS * ROWS_PER_TILE
HSP_PER_TILE = 3128
N_HSP = NS * HSP_PER_TILE

PR = 1568
N_POOL = NW * PR
GP = G + 1

BLK = 2000
NBLK = N // BLK

_MESH = plsc.VectorSubcoreMesh(
    core_axis_name="c", subcore_axis_name="s", num_cores=NC, num_subcores=NS
)
_SC_PARAMS = pltpu.CompilerParams(use_tc_tiling_on_sc=False)


def _make_sc_agg(dh, nhalf):
    ncw_half = NCW // 2

    @functools.partial(
        pl.kernel,
        out_type=jax.ShapeDtypeStruct((nhalf, NC, N_ACC, dh), jnp.float32),
        mesh=_MESH,
        compiler_params=_SC_PARAMS,
        scratch_types=[
            pltpu.VMEM((KC, CHUNK), jnp.int32),
            pltpu.VMEM((KC, CHUNK), jnp.int32),
            pltpu.VMEM((KC, CHUNK), jnp.int32),
            pltpu.VMEM((KC, CHUNK), jnp.int32),
            pltpu.VMEM((CHUNK, dh), jnp.float32),
            pltpu.VMEM((CHUNK, dh), jnp.float32),
            pltpu.VMEM((CHUNK, dh), jnp.float32),
            pltpu.VMEM((CHUNK, dh), jnp.float32),
            pltpu.VMEM_SHARED((N_HSP, dh), jnp.float32),
            pltpu.VMEM_SHARED((N_ACC, dh), jnp.float32),
            pltpu.SemaphoreType.DMA,
            pltpu.SemaphoreType.DMA,
            pltpu.SemaphoreType.DMA,
            pltpu.SemaphoreType.DMA,
            pltpu.SemaphoreType.DMA,
            pltpu.SemaphoreType.DMA,
            pltpu.SemaphoreType.DMA,
            pltpu.SemaphoreType.DMA,
            pltpu.SemaphoreType.DMA,
        ],
    )
    def agg(h0_hbm, h1_hbm, src_hbm, dst_hbm, zer_hbm, out_hbm,
            src_vA, dst_vA, src_vB, dst_vB, b0, b1, b2, b3, h_sp, acc,
            g0s, g1s, g2s, g3s, s0s, s1s, s2s, s3s, isem):
        cid = lax.axis_index("c")
        sid = lax.axis_index("s")
        cbase = cid * NS * ncw_half + sid * ncw_half
        bufs = (b0, b1, b2, b3)
        gsems = (g0s, g1s, g2s, g3s)
        ssems = (s0s, s1s, s2s, s3s)
        myrows = pl.ds(sid * ROWS_PER_TILE, ROWS_PER_TILE)
        hrows = pl.ds(sid * HSP_PER_TILE, HSP_PER_TILE)
        halves = (h0_hbm, h1_hbm)

        def phase(k):
            pltpu.sync_copy(halves[k].at[hrows, :], h_sp.at[hrows, :])
            pltpu.sync_copy(zer_hbm, acc.at[myrows, :])
            plsc.subcore_barrier()

            def process_chunks(src_v, dst_v):
                for b in range(4):
                    pltpu.async_copy(h_sp.at[src_v.at[b]], bufs[b], gsems[b])

                def qbody(t, _):
                    jb = 4 * t
                    for b in range(4):
                        j = jb + b
                        pltpu.make_async_copy(
                            h_sp.at[src_v.at[j]], bufs[b], gsems[b]
                        ).wait()
                        pltpu.async_copy(
                            bufs[b], acc.at[dst_v.at[j]], ssems[b], add=True
                        )
                    for b in range(4):
                        nj = jb + b + 4

                        @pl.when(nj < KC)
                        def _(b=b, nj=nj, jb=jb):
                            pltpu.make_async_copy(
                                bufs[b], acc.at[dst_v.at[jb + b]], ssems[b]
                            ).wait()
                            pltpu.async_copy(
                                h_sp.at[src_v.at[nj]], bufs[b], gsems[b]
                            )

                    return 0

                lax.fori_loop(0, KC // 4, qbody, 0)
                for b in range(4):
                    pltpu.make_async_copy(
                        bufs[b], acc.at[dst_v.at[KC - 4 + b]], ssems[b]
                    ).wait()

            def stage(g, sv, dv, sem):
                pltpu.async_copy(src_hbm.at[pl.ds(cbase + g * KC, KC)], sv, sem)
                pltpu.async_copy(dst_hbm.at[pl.ds(cbase + g * KC, KC)], dv, sem)

            def stage_wait(g, sv, dv, sem):
                pltpu.make_async_copy(
                    src_hbm.at[pl.ds(cbase + g * KC, KC)], sv, sem
                ).wait()
                pltpu.make_async_copy(
                    dst_hbm.at[pl.ds(cbase + g * KC, KC)], dv, sem
                ).wait()

            pltpu.sync_copy(src_hbm.at[pl.ds(cbase, KC)], src_vA)
            pltpu.sync_copy(dst_hbm.at[pl.ds(cbase, KC)], dst_vA)

            ng = ncw_half // KC

            def gpair(q, _):
                ga = 2 * q
                stage(ga + 1, src_vB, dst_vB, isem)
                process_chunks(src_vA, dst_vA)
                stage_wait(ga + 1, src_vB, dst_vB, isem)

                @pl.when(ga + 2 < ng)
                def _():
                    stage(ga + 2, src_vA, dst_vA, isem)

                process_chunks(src_vB, dst_vB)

                @pl.when(ga + 2 < ng)
                def _():
                    stage_wait(ga + 2, src_vA, dst_vA, isem)

                return 0

            lax.fori_loop(0, ng // 2, gpair, 0)
            plsc.subcore_barrier()

            pltpu.sync_copy(acc.at[myrows, :], out_hbm.at[k, cid, myrows, :])
            plsc.subcore_barrier()

        for k in range(nhalf):
            phase(k)

    return agg


_sc_agg8 = _make_sc_agg(8, 1)
_sc_agg32 = _make_sc_agg(DH, 2)


@functools.partial(
    pl.kernel,
    out_type=jax.ShapeDtypeStruct((NW, GP, D1), jnp.float32),
    mesh=_MESH,
    compiler_params=_SC_PARAMS,
    scratch_types=[
        pltpu.VMEM((PR,), jnp.int32),
        pltpu.VMEM((PR, D1), jnp.float32),
        pltpu.VMEM((GP, D1), jnp.float32),
    ],
)
def _sc_pool(h_hbm, b_hbm, out_hbm, batch_v, h_v, acc):
    cid = lax.axis_index("c")
    sid = lax.axis_index("s")
    wid = cid * NS + sid
    base = wid * PR
    pltpu.sync_copy(b_hbm.at[pl.ds(base, PR)], batch_v)
    pltpu.sync_copy(h_hbm.at[pl.ds(base, PR), :], h_v)

    neg = jnp.full((L,), -3.4e38, jnp.float32)

    def ini(g, _):
        acc[g, pl.ds(0, L)] = neg
        acc[g, pl.ds(L, L)] = neg
        return 0

    lax.fori_loop(0, GP, ini, 0)

    def body(q, _):
        gvec = batch_v[pl.ds(q * L, L)]
        for lane in range(L):
            r = q * L + lane
            g = gvec[lane]
            acc[g, pl.ds(0, L)] = jnp.maximum(
                acc[g, pl.ds(0, L)], h_v[r, pl.ds(0, L)]
            )
            acc[g, pl.ds(L, L)] = jnp.maximum(
                acc[g, pl.ds(L, L)], h_v[r, pl.ds(L, L)]
            )
        return 0

    lax.fori_loop(0, PR // L, body, 0)
    pltpu.sync_copy(acc, out_hbm.at[wid])


def _make_layer(d, nhalf):
    dh = d // nhalf
    nh_in = 2 if d == D1 else 1

    def body(*refs):
        h_refs = refs[:nh_in]
        agg_ref, w1, b1, w2, b2, t_ref, st_ref, accs = refs[nh_in:]
        i = pl.program_id(0)
        if nh_in == 2:
            h = jnp.concatenate([h_refs[0][...], h_refs[1][...]], axis=-1)
        else:
            h = h_refs[0][...]
        a = agg_ref[0, 0] + agg_ref[0, 1]
        if nhalf == 2:
            a = jnp.concatenate([a, agg_ref[1, 0] + agg_ref[1, 1]], axis=-1)
        u = h + a
        t = jnp.dot(u, w1[...], preferred_element_type=jnp.float32) + b1[...]
        t = jnp.maximum(t, 0.0)
        t = jnp.dot(t, w2[...], preferred_element_type=jnp.float32) + b2[...]
        t = jnp.maximum(t, 0.0)
        t_ref[...] = t

        @pl.when(i == 0)
        def _():
            accs[...] = jnp.zeros_like(accs)

        s = jnp.sum(t, axis=0, keepdims=True)
        m = s * (1.0 / BLK)
        d2 = t - m
        accs[0:1, :] += s
        accs[1:2, :] += jnp.sum(d2 * d2, axis=0, keepdims=True)
        accs[2:3, :] += m * m

        @pl.when(i == NBLK - 1)
        def _():
            st_ref[...] = accs[...]

    h_specs = [
        pl.BlockSpec((BLK, d // nh_in), lambda i: (i, 0)) for _ in range(nh_in)
    ]
    return pl.pallas_call(
        body,
        grid=(NBLK,),
        in_specs=h_specs
        + [
            pl.BlockSpec((nhalf, NC, BLK, dh), lambda i: (0, 0, i, 0)),
            pl.BlockSpec((d, D1), lambda i: (0, 0)),
            pl.BlockSpec((1, D1), lambda i: (0, 0)),
            pl.BlockSpec((D1, D1), lambda i: (0, 0)),
            pl.BlockSpec((1, D1), lambda i: (0, 0)),
        ],
        out_specs=[
            pl.BlockSpec((BLK, D1), lambda i: (i, 0)),
            pl.BlockSpec((8, D1), lambda i: (0, 0)),
        ],
        out_shape=[
            jax.ShapeDtypeStruct((N, D1), jnp.float32),
            jax.ShapeDtypeStruct((8, D1), jnp.float32),
        ],
        scratch_shapes=[pltpu.VMEM((8, D1), jnp.float32)],
    )


_layer8 = _make_layer(8, 1)
_layer32 = _make_layer(D1, 2)


def _make_norm_halves():
    def body(t_ref, m_ref, s_ref, c_ref, o0_ref, o1_ref):
        hh = (t_ref[...] - m_ref[...]) * s_ref[...] + c_ref[...]
        o0_ref[...] = hh[:, :DH]
        o1_ref[...] = hh[:, DH:]

    return pl.pallas_call(
        body,
        grid=(NBLK,),
        in_specs=[
            pl.BlockSpec((BLK, D1), lambda i: (i, 0)),
            pl.BlockSpec((1, D1), lambda i: (0, 0)),
            pl.BlockSpec((1, D1), lambda i: (0, 0)),
            pl.BlockSpec((1, D1), lambda i: (0, 0)),
        ],
        out_specs=[
            pl.BlockSpec((BLK, DH), lambda i: (i, 0)),
            pl.BlockSpec((BLK, DH), lambda i: (i, 0)),
        ],
        out_shape=[
            jax.ShapeDtypeStruct((N_HSP, DH), jnp.float32),
            jax.ShapeDtypeStruct((N_HSP, DH), jnp.float32),
        ],
    )


_norm_halves = _make_norm_halves()


def _make_norm_pool():
    def body(t_ref, m_ref, s_ref, c_ref, o_ref):
        o_ref[...] = (t_ref[...] - m_ref[...]) * s_ref[...] + c_ref[...]

    return pl.pallas_call(
        body,
        grid=(NBLK,),
        in_specs=[
            pl.BlockSpec((BLK, D1), lambda i: (i, 0)),
            pl.BlockSpec((1, D1), lambda i: (0, 0)),
            pl.BlockSpec((1, D1), lambda i: (0, 0)),
            pl.BlockSpec((1, D1), lambda i: (0, 0)),
        ],
        out_specs=pl.BlockSpec((BLK, D1), lambda i: (i, 0)),
        out_shape=jax.ShapeDtypeStruct((N_POOL, D1), jnp.float32),
    )


_norm_pool = _make_norm_pool()


def _head(pmb, pmr, wb, bb, wr, br2, wbe, bbe, wm, bm):
    def body(pmb_ref, pmr_ref, wb_r, bb_r, wr_r, br_r, wbe_r, bbe_r, wm_r, bm_r,
             o_ref):
        pb = jnp.max(pmb_ref[...], axis=0)[:G, :]
        eb = jnp.maximum(
            jnp.dot(pb, wb_r[...], preferred_element_type=jnp.float32) + bb_r[...], 0.0
        )
        pr = jnp.max(pmr_ref[...], axis=0)[:G, :]
        er = jnp.maximum(
            jnp.dot(pr, wr_r[...], preferred_element_type=jnp.float32) + br_r[...], 0.0
        )
        cat = jnp.concatenate([eb, er], axis=-1)
        h = jnp.maximum(
            jnp.dot(cat, wbe_r[...], preferred_element_type=jnp.float32) + bbe_r[...],
            0.0,
        )
        z = jnp.dot(h, wm_r[...], preferred_element_type=jnp.float32) + bm_r[...]
        o_ref[...] = 1.0 / (1.0 + jnp.exp(-z))

    return pl.pallas_call(
        body,
        out_shape=jax.ShapeDtypeStruct((G, 1), jnp.float32),
    )(pmb, pmr, wb, bb, wr, br2, wbe, bbe, wm, bm)


def _prep_edges(ei):
    src = jnp.concatenate([ei[0], jnp.zeros((E_PAD - E,), jnp.int32)])
    pad_dst = N + (jnp.arange(E_PAD - E, dtype=jnp.int32) % (N_ACC - N))
    dst = jnp.concatenate([ei[1], pad_dst])
    return src.reshape(TOT_CH, CHUNK), dst.reshape(TOT_CH, CHUNK)


def _branch(x, ei, batch, br, p):
    srcr, dstr = _prep_edges(ei)
    batch_p = jnp.pad(batch, (0, N_POOL - N), constant_values=G).astype(jnp.int32)
    zer8 = jnp.zeros((ROWS_PER_TILE, 8), jnp.float32)
    zer16 = jnp.zeros((ROWS_PER_TILE, DH), jnp.float32)

    x8 = jnp.pad(x, ((0, N_HSP - N), (0, 8 - x.shape[1])))
    h_halves = (x8, x8)
    for i in range(1, 4):
        d = 8 if i == 1 else D1
        w1 = p[br + "_c%d_W1" % i]
        if i == 1:
            w1 = jnp.pad(w1, ((0, 8 - w1.shape[0]), (0, 0)))
        agg = (_sc_agg8 if d == 8 else _sc_agg32)(
            h_halves[0], h_halves[1], srcr, dstr, zer8 if d == 8 else zer16
        )
        h_in = (h_halves[0],) if d == 8 else h_halves
        t, st = (_layer8 if d == 8 else _layer32)(
            *h_in, agg, w1,
            p[br + "_c%d_b1" % i].reshape(1, D1),
            p[br + "_c%d_W2" % i],
            p[br + "_c%d_b2" % i].reshape(1, D1),
        )
        mean = st[0] / N
        var = st[1] / N + (st[2] / NBLK - mean * mean)
        s = p[br + "_bn%d_g" % i] / jnp.sqrt(var + 1e-5)
        m = mean.reshape(1, D1)
        s = s.reshape(1, D1)
        c = p[br + "_bn%d_b" % i].reshape(1, D1)
        if i < 3:
            h_halves = _norm_halves(t, m, s, c)
        else:
            h3 = _norm_pool(t, m, s, c)

    return _sc_pool(h3, batch_p)


def kernel(data_base, edge_index_base, batch_base,
           data_residual, edge_index_residual, batch_residual, params):
    p = params
    pmb = _branch(data_base, edge_index_base, batch_base, "base", p)
    pmr = _branch(data_residual, edge_index_residual, batch_residual, "res", p)
    return _head(
        pmb, pmr,
        p["base_Wbr"], p["base_bbr"].reshape(1, D1),
        p["res_Wbr"], p["res_bbr"].reshape(1, D1),
        p["W_before"], p["b_before"].reshape(1, 16),
        p["W_mean"], p["b_mean"].reshape(1, 1),
    )
